# baseline (speedup 1.0000x reference)
import functools

import jax
import jax.numpy as jnp
from jax import lax
from jax.experimental import pallas as pl
from jax.experimental.pallas import tpu as pltpu
from jax.experimental.pallas import tpu_sc as plsc

N = 10000
E = 320000
F = 128
C = 40

NC, NS = 2, 16
NW = NC * NS
E_PER_W = E // NW
K = 80
CHUNKS = E_PER_W // K
KS = 80
CHUNKS_S = E_PER_W // KS
NBS = 3
MAIN_S = (CHUNKS_S // NBS) * NBS
RPT = N // NS
ZR = 25

_HIGH = lax.Precision.HIGHEST


@functools.cache
def _mesh():
    return plsc.VectorSubcoreMesh(core_axis_name="c", subcore_axis_name="s",
                                  num_cores=NC, num_subcores=NS)


def _zero_fill(zbuf, nrows, ncols):
    zero16 = jnp.zeros((16,), jnp.float32)

    def body(r, carry):
        for j in range(ncols // 16):
            zbuf[r, pl.ds(j * 16, 16)] = zero16
        return carry

    lax.fori_loop(0, nrows, body, 0)


NPAD = 10240
RPT1 = NPAD // NS


NB = 5


def _deg_body(dst3, out_hbm, acc, didx2, ones_v, zbuf, *sems):
    c = lax.axis_index("c")
    s = lax.axis_index("s")
    w = c * NS + s
    zero16 = jnp.zeros((16,), jnp.float32)
    ones16 = jnp.ones((16,), jnp.float32)
    for i in range(RPT1 // 16):
        zbuf[pl.ds(i * 16, 16)] = zero16
    for i in range(K // 16):
        ones_v[pl.ds(i * 16, 16)] = ones16
    pltpu.sync_copy(dst3.at[w], didx2)
    pltpu.sync_copy(zbuf, acc.at[pl.ds(s * RPT1, RPT1)])
    plsc.subcore_barrier()

    for b in range(NB):
        pltpu.async_copy(ones_v, acc.at[didx2.at[b]], sems[b], add=True)

    def body(t, carry):
        for b in range(NB):
            j = t * NB + b
            pltpu.make_async_copy(ones_v, acc.at[didx2.at[j]], sems[b]).wait()
            pltpu.async_copy(ones_v, acc.at[didx2.at[j + NB]], sems[b],
                             add=True)
        return carry

    lax.fori_loop(0, CHUNKS // NB - 1, body, 0)
    for b in range(NB):
        pltpu.make_async_copy(ones_v, acc.at[didx2.at[b]], sems[b]).wait()
    plsc.subcore_barrier()
    pltpu.sync_copy(acc.at[pl.ds(s * RPT1, RPT1)], out_hbm.at[c, s])


@functools.cache
def _deg_call():
    return pl.kernel(
        _deg_body,
        out_type=jax.ShapeDtypeStruct((NC, NS, RPT1), jnp.float32),
        mesh=_mesh(),
        scratch_types=[
            pltpu.VMEM_SHARED((NPAD,), jnp.float32),
            pltpu.VMEM((CHUNKS, K), jnp.int32),
            pltpu.VMEM((K,), jnp.float32),
            pltpu.VMEM((RPT1,), jnp.float32),
        ] + [pltpu.SemaphoreType.DMA] * NB,
    )


def _scat_body(src2, dst3, g_hbm, out_hbm, acc, sidx1, zbuf, *rest):
    rows = rest[:NBS]
    didx = rest[NBS:2 * NBS]
    gsem = rest[2 * NBS:3 * NBS]
    ssem = rest[3 * NBS:4 * NBS]
    dsem = rest[4 * NBS:5 * NBS]
    c = lax.axis_index("c")
    s = lax.axis_index("s")
    w = c * NS + s
    _zero_fill(zbuf, ZR, F)
    pltpu.sync_copy(src2.at[w], sidx1)
    for j in range(RPT // ZR):
        pltpu.sync_copy(zbuf, acc.at[pl.ds(s * RPT + j * ZR, ZR)])
    plsc.subcore_barrier()

    def start_front(j, b):
        pltpu.async_copy(g_hbm.at[sidx1.at[pl.ds(j * KS, KS)]], rows[b],
                         gsem[b])
        pltpu.async_copy(dst3.at[w, pl.ds(j, 1)], didx[b], dsem[b])

    def wait_front(b):
        pltpu.make_async_copy(g_hbm.at[sidx1.at[pl.ds(0, KS)]], rows[b],
                              gsem[b]).wait()
        pltpu.make_async_copy(dst3.at[w, pl.ds(0, 1)], didx[b],
                              dsem[b]).wait()

    def start_scat(b):
        pltpu.async_copy(rows[b], acc.at[didx[b].at[0]], ssem[b], add=True)

    def wait_scat(b):
        pltpu.make_async_copy(rows[b], acc.at[didx[b].at[0]], ssem[b]).wait()

    for b in range(NBS):
        start_front(b, b)

    def body(t, carry):
        for b in range(NBS):
            wait_front(b)
            start_scat(b)
        for b in range(NBS):
            j = t * NBS + b
            jn = jnp.minimum(j + NBS, CHUNKS_S - 1)
            wait_scat(b)
            start_front(jn, b)
        return carry

    lax.fori_loop(0, MAIN_S // NBS, body, 0)
    for b in range(NBS):
        wait_front(b)
    for j in range(MAIN_S, CHUNKS_S):
        b = j - MAIN_S
        start_front(jnp.int32(j), b)
        wait_front(b)
        start_scat(b)
        wait_scat(b)
    plsc.subcore_barrier()
    pltpu.sync_copy(acc.at[pl.ds(s * RPT, RPT)], out_hbm.at[c, s])


@functools.cache
def _scat_call():
    return pl.kernel(
        _scat_body,
        out_type=jax.ShapeDtypeStruct((NC, NS, RPT, F), jnp.float32),
        mesh=_mesh(),
        scratch_types=[
            pltpu.VMEM_SHARED((N, F), jnp.float32),
            pltpu.VMEM((E_PER_W,), jnp.int32),
            pltpu.VMEM((ZR, F), jnp.float32),
        ] + [pltpu.VMEM((KS, F), jnp.float32)] * NBS
          + [pltpu.VMEM((1, KS), jnp.int32)] * NBS
          + [pltpu.SemaphoreType.DMA] * (3 * NBS),
    )


BLK = 1000
GRID = N // BLK


def _tc_a_body(degp, x, w0, g0, dis_o):
    deg = degp[0, :, 0] + degp[1, :, 0] + 1.0
    dis = lax.rsqrt(jnp.maximum(deg, 1.0))[:, None]
    g0[...] = jnp.dot(x[...], w0[...], preferred_element_type=jnp.float32,
                      precision=_HIGH) * dis
    dis_o[...] = dis


def _tc_b_body(p, gprev, dis_r, b, w, h_o, gnext_o):
    dis = dis_r[...]
    h = jnp.maximum((p[0] + p[1] + gprev[...]) * dis + b[...], 0.0)
    h_o[...] = h
    gnext_o[...] = jnp.dot(h, w[...], preferred_element_type=jnp.float32,
                           precision=_HIGH) * dis


def _tc_c_body(p, g2, dis_r, b2, h1, h2, wout, bout, out_o):
    dis = dis_r[...]
    h3 = jnp.maximum((p[0] + p[1] + g2[...]) * dis + b2[...], 0.0)
    hjk = jnp.maximum(jnp.maximum(h1[...], h2[...]), h3)
    out_o[...] = jnp.dot(hjk, wout[...], preferred_element_type=jnp.float32,
                         precision=_HIGH) + bout[...]


def _rows(blk, w):
    return pl.BlockSpec((blk, w), lambda i: (i, 0))


def _whole(shape):
    return pl.BlockSpec(shape, lambda i: tuple(0 for _ in shape))


_tc_a = pl.pallas_call(
    _tc_a_body,
    grid=(GRID,),
    in_specs=[pl.BlockSpec((NC, BLK, 1), lambda i: (0, i, 0)),
              _rows(BLK, F), _whole((F, F))],
    out_specs=[_rows(BLK, F), _rows(BLK, 1)],
    out_shape=[jax.ShapeDtypeStruct((N, F), jnp.float32),
               jax.ShapeDtypeStruct((N, 1), jnp.float32)],
)

_tc_b = pl.pallas_call(
    _tc_b_body,
    grid=(GRID,),
    in_specs=[pl.BlockSpec((NC, BLK, F), lambda i: (0, i, 0)),
              _rows(BLK, F), _rows(BLK, 1), _whole((1, F)), _whole((F, F))],
    out_specs=[_rows(BLK, F), _rows(BLK, F)],
    out_shape=[jax.ShapeDtypeStruct((N, F), jnp.float32),
               jax.ShapeDtypeStruct((N, F), jnp.float32)],
)

_tc_c = pl.pallas_call(
    _tc_c_body,
    grid=(GRID,),
    in_specs=[pl.BlockSpec((NC, BLK, F), lambda i: (0, i, 0)),
              _rows(BLK, F), _rows(BLK, 1), _whole((1, F)),
              _rows(BLK, F), _rows(BLK, F), _whole((F, C)), _whole((1, C))],
    out_specs=_rows(BLK, C),
    out_shape=jax.ShapeDtypeStruct((N, C), jnp.float32),
)


def kernel(x, edge_index, W0, b0, W1, b1, W2, b2, Wout, bout):
    src2 = edge_index[0].reshape(NW, E_PER_W)
    dst3 = edge_index[1].reshape(NW, CHUNKS_S, KS)
    dst3d = edge_index[1].reshape(NW, CHUNKS, K)
    degp = _deg_call()(dst3d).reshape(NC, NPAD)[:, :N].reshape(NC, N, 1)
    g0, dis = _tc_a(degp, x, W0)
    p0 = _scat_call()(src2, dst3, g0).reshape(NC, N, F)
    h1, g1 = _tc_b(p0, g0, dis, b0.reshape(1, F), W1)
    p1 = _scat_call()(src2, dst3, g1).reshape(NC, N, F)
    h2, g2 = _tc_b(p1, g1, dis, b1.reshape(1, F), W2)
    p2 = _scat_call()(src2, dst3, g2).reshape(NC, N, F)
    return _tc_c(p2, g2, dis, b2.reshape(1, F), h1, h2, Wout,
                 bout.reshape(1, C))


# speedup vs baseline: 1.0610x; 1.0610x over previous
import functools

import jax
import jax.numpy as jnp
from jax import lax
from jax.experimental import pallas as pl
from jax.experimental.pallas import tpu as pltpu
from jax.experimental.pallas import tpu_sc as plsc

N = 10000
E = 320000
F = 128
C = 40

NC, NS = 2, 16
NW = NC * NS
E_PER_W = E // NW
K = 80
CHUNKS = E_PER_W // K
KS = 40
CHUNKS_S = E_PER_W // KS
RPT = N // NS
ZR = 75

_HIGH = lax.Precision.HIGHEST


@functools.cache
def _mesh():
    return plsc.VectorSubcoreMesh(core_axis_name="c", subcore_axis_name="s",
                                  num_cores=NC, num_subcores=NS)


def _zero_fill(zbuf, nrows, ncols):
    zero16 = jnp.zeros((16,), jnp.float32)

    def body(r, carry):
        for j in range(ncols // 16):
            zbuf[r, pl.ds(j * 16, 16)] = zero16
        return carry

    lax.fori_loop(0, nrows, body, 0)


NPAD = 10240
RPT1 = NPAD // NS


NB = 5


def _deg_body(dst3, out_hbm, acc, didx2, ones_v, zbuf, *sems):
    c = lax.axis_index("c")
    s = lax.axis_index("s")
    w = c * NS + s
    zero16 = jnp.zeros((16,), jnp.float32)
    ones16 = jnp.ones((16,), jnp.float32)
    for i in range(RPT1 // 16):
        zbuf[pl.ds(i * 16, 16)] = zero16
    for i in range(K // 16):
        ones_v[pl.ds(i * 16, 16)] = ones16
    pltpu.sync_copy(dst3.at[w], didx2)
    pltpu.sync_copy(zbuf, acc.at[pl.ds(s * RPT1, RPT1)])
    plsc.subcore_barrier()

    for b in range(NB):
        pltpu.async_copy(ones_v, acc.at[didx2.at[b]], sems[b], add=True)

    def body(t, carry):
        for b in range(NB):
            j = t * NB + b
            pltpu.make_async_copy(ones_v, acc.at[didx2.at[j]], sems[b]).wait()
            pltpu.async_copy(ones_v, acc.at[didx2.at[j + NB]], sems[b],
                             add=True)
        return carry

    lax.fori_loop(0, CHUNKS // NB - 1, body, 0)
    for b in range(NB):
        pltpu.make_async_copy(ones_v, acc.at[didx2.at[b]], sems[b]).wait()
    plsc.subcore_barrier()
    pltpu.sync_copy(acc.at[pl.ds(s * RPT1, RPT1)], out_hbm.at[c, s])


@functools.cache
def _deg_call():
    return pl.kernel(
        _deg_body,
        out_type=jax.ShapeDtypeStruct((NC, NS, RPT1), jnp.float32),
        mesh=_mesh(),
        scratch_types=[
            pltpu.VMEM_SHARED((NPAD,), jnp.float32),
            pltpu.VMEM((CHUNKS, K), jnp.int32),
            pltpu.VMEM((K,), jnp.float32),
            pltpu.VMEM((RPT1,), jnp.float32),
        ] + [pltpu.SemaphoreType.DMA] * NB,
    )


def _scat_body(src2, dst3, g_hbm, out_hbm, acc, sidx1, zbuf, *rest):
    rows = rest[:NB]
    didx = rest[NB:2 * NB]
    gsem = rest[2 * NB:3 * NB]
    ssem = rest[3 * NB:4 * NB]
    dsem = rest[4 * NB:5 * NB]
    c = lax.axis_index("c")
    s = lax.axis_index("s")
    w = c * NS + s
    _zero_fill(zbuf, ZR, F)
    pltpu.sync_copy(src2.at[w], sidx1)
    for j in range(RPT // ZR):
        pltpu.sync_copy(zbuf, acc.at[pl.ds(s * RPT + j * ZR, ZR)])
    pltpu.sync_copy(zbuf.at[pl.ds(0, RPT - (RPT // ZR) * ZR)],
                    acc.at[pl.ds(s * RPT + (RPT // ZR) * ZR,
                                 RPT - (RPT // ZR) * ZR)])
    plsc.subcore_barrier()

    def start_front(j, b):
        pltpu.async_copy(g_hbm.at[sidx1.at[pl.ds(j * KS, KS)]], rows[b],
                         gsem[b])
        pltpu.async_copy(dst3.at[w, j], didx[b], dsem[b])

    for b in range(NB):
        start_front(b, b)

    def body(t, carry):
        for b in range(NB):
            j = t * NB + b
            pltpu.make_async_copy(g_hbm.at[sidx1.at[pl.ds(j * KS, KS)]],
                                  rows[b], gsem[b]).wait()
            pltpu.make_async_copy(dst3.at[w, j], didx[b], dsem[b]).wait()
            pltpu.async_copy(rows[b], acc.at[didx[b]], ssem[b], add=True)
        for b in range(NB):
            j = t * NB + b
            jn = jnp.minimum(j + NB, CHUNKS_S - 1)
            pltpu.make_async_copy(rows[b], acc.at[didx[b]], ssem[b]).wait()
            start_front(jn, b)
        return carry

    lax.fori_loop(0, CHUNKS_S // NB, body, 0)
    for b in range(NB):
        pltpu.make_async_copy(g_hbm.at[sidx1.at[pl.ds(0, KS)]], rows[b],
                              gsem[b]).wait()
        pltpu.make_async_copy(dst3.at[w, 0], didx[b], dsem[b]).wait()
    plsc.subcore_barrier()
    pltpu.sync_copy(acc.at[pl.ds(s * RPT, RPT)], out_hbm.at[c, s])


@functools.cache
def _scat_call():
    return pl.kernel(
        _scat_body,
        out_type=jax.ShapeDtypeStruct((NC, NS, RPT, F), jnp.float32),
        mesh=_mesh(),
        scratch_types=[
            pltpu.VMEM_SHARED((N, F), jnp.float32),
            pltpu.VMEM((E_PER_W,), jnp.int32),
            pltpu.VMEM((ZR, F), jnp.float32),
        ] + [pltpu.VMEM((KS, F), jnp.float32)] * NB
          + [pltpu.VMEM((KS,), jnp.int32)] * NB
          + [pltpu.SemaphoreType.DMA] * (3 * NB),
    )


BLK = 1000
GRID = N // BLK


def _tc_a_body(degp, x, w0, g0, dis_o):
    deg = degp[0, :, 0] + degp[1, :, 0] + 1.0
    dis = lax.rsqrt(jnp.maximum(deg, 1.0))[:, None]
    g0[...] = jnp.dot(x[...], w0[...], preferred_element_type=jnp.float32,
                      precision=_HIGH) * dis
    dis_o[...] = dis


def _tc_b_body(p, gprev, dis_r, b, w, h_o, gnext_o):
    dis = dis_r[...]
    h = jnp.maximum((p[0] + p[1] + gprev[...]) * dis + b[...], 0.0)
    h_o[...] = h
    gnext_o[...] = jnp.dot(h, w[...], preferred_element_type=jnp.float32,
                           precision=_HIGH) * dis


def _tc_c_body(p, g2, dis_r, b2, h1, h2, wout, bout, out_o):
    dis = dis_r[...]
    h3 = jnp.maximum((p[0] + p[1] + g2[...]) * dis + b2[...], 0.0)
    hjk = jnp.maximum(jnp.maximum(h1[...], h2[...]), h3)
    out_o[...] = jnp.dot(hjk, wout[...], preferred_element_type=jnp.float32,
                         precision=_HIGH) + bout[...]


def _rows(blk, w):
    return pl.BlockSpec((blk, w), lambda i: (i, 0))


def _whole(shape):
    return pl.BlockSpec(shape, lambda i: tuple(0 for _ in shape))


_tc_a = pl.pallas_call(
    _tc_a_body,
    grid=(GRID,),
    in_specs=[pl.BlockSpec((NC, BLK, 1), lambda i: (0, i, 0)),
              _rows(BLK, F), _whole((F, F))],
    out_specs=[_rows(BLK, F), _rows(BLK, 1)],
    out_shape=[jax.ShapeDtypeStruct((N, F), jnp.float32),
               jax.ShapeDtypeStruct((N, 1), jnp.float32)],
)

_tc_b = pl.pallas_call(
    _tc_b_body,
    grid=(GRID,),
    in_specs=[pl.BlockSpec((NC, BLK, F), lambda i: (0, i, 0)),
              _rows(BLK, F), _rows(BLK, 1), _whole((1, F)), _whole((F, F))],
    out_specs=[_rows(BLK, F), _rows(BLK, F)],
    out_shape=[jax.ShapeDtypeStruct((N, F), jnp.float32),
               jax.ShapeDtypeStruct((N, F), jnp.float32)],
)

_tc_c = pl.pallas_call(
    _tc_c_body,
    grid=(GRID,),
    in_specs=[pl.BlockSpec((NC, BLK, F), lambda i: (0, i, 0)),
              _rows(BLK, F), _rows(BLK, 1), _whole((1, F)),
              _rows(BLK, F), _rows(BLK, F), _whole((F, C)), _whole((1, C))],
    out_specs=_rows(BLK, C),
    out_shape=jax.ShapeDtypeStruct((N, C), jnp.float32),
)


def kernel(x, edge_index, W0, b0, W1, b1, W2, b2, Wout, bout):
    src2 = edge_index[0].reshape(NW, E_PER_W)
    dst3 = edge_index[1].reshape(NW, CHUNKS_S, KS)
    dst3d = edge_index[1].reshape(NW, CHUNKS, K)
    degp = _deg_call()(dst3d).reshape(NC, NPAD)[:, :N].reshape(NC, N, 1)
    g0, dis = _tc_a(degp, x, W0)
    p0 = _scat_call()(src2, dst3, g0).reshape(NC, N, F)
    h1, g1 = _tc_b(p0, g0, dis, b0.reshape(1, F), W1)
    p1 = _scat_call()(src2, dst3, g1).reshape(NC, N, F)
    h2, g2 = _tc_b(p1, g1, dis, b1.reshape(1, F), W2)
    p2 = _scat_call()(src2, dst3, g2).reshape(NC, N, F)
    return _tc_c(p2, g2, dis, b2.reshape(1, F), h1, h2, Wout,
                 bout.reshape(1, C))
